# baseline (speedup 1.0000x reference)
import functools

import jax
import jax.numpy as jnp
from jax import lax
from jax.experimental import pallas as pl
from jax.experimental.pallas import tpu as pltpu
from jax.experimental.pallas import tpu_sc as plsc

_B, _K, _D, _E, _P = 128, 36, 2048, 1024, 5
_GB = 4
_RT = _GB * _K
_N = _B * _K

_NC, _NS, _L = 2, 16, 16
_NW = _NC * _NS
_SLABW = 128
_NSLAB = _N // _SLABW
_BLKS = _SLABW // _L


def _sc_relation_weights(rng_hbm, w_hbm, buf, sem):
    f32 = jnp.float32
    i32 = jnp.int32
    wid = lax.axis_index("s") * _NC + lax.axis_index("c")
    lanes = lax.iota(i32, _L)

    def do_slab(slab):
        base = slab * _SLABW
        cp = pltpu.make_async_copy(rng_hbm.at[:, pl.ds(base, _SLABW)],
                                   buf, sem)
        cp.start()
        cp.wait()

        def blk_body(blk, carry):
            col0 = blk * _L
            run = jnp.zeros((_L,), f32)
            for j in range(_K):
                c = buf[j, pl.ds(col0, _L)]
                run = run + c
                buf[j, pl.ds(col0, _L)] = jnp.where(run <= float(_P), c, 0.0)
            deficit = jnp.maximum(float(_P) - run, 0.0)
            dcol = jnp.remainder(base + col0 + lanes, _K)
            for j in range(_K):
                wv = buf[j, pl.ds(col0, _L)]
                buf[j, pl.ds(col0, _L)] = wv + jnp.where(dcol == j,
                                                         deficit, 0.0)
            return carry

        lax.fori_loop(0, _BLKS, blk_body, 0)

        cp = pltpu.make_async_copy(buf, w_hbm.at[:, pl.ds(base, _SLABW)], sem)
        cp.start()
        cp.wait()

    do_slab(wid)

    @pl.when(wid < _NSLAB - _NW)
    def _second_slab():
        do_slab(wid + _NW)


_sc_w_kernel = functools.partial(
    pl.kernel,
    mesh=plsc.VectorSubcoreMesh(core_axis_name="c", subcore_axis_name="s"),
    out_type=jax.ShapeDtypeStruct((_K, _N), jnp.float32),
    scratch_types=[
        pltpu.VMEM((_K, _SLABW), jnp.float32),
        pltpu.SemaphoreType.DMA,
    ],
)(_sc_relation_weights)


def _tc_body(imgs_ref, s_ref, w_ref,
             gw1a_ref, gw1b_ref, gb1_ref, gw2_ref, gb2_ref,
             nw1a_ref, nw1b_ref, nb1_ref, nw2_ref, nb2_ref,
             mw1_ref, mb1_ref, mw2_ref, mb2_ref,
             out_ref):
    f32 = jnp.float32
    bf16 = jnp.bfloat16

    xb = imgs_ref[...]
    xb16 = xb.astype(bf16)
    sb16 = s_ref[...].astype(bf16)

    hg = jnp.dot(xb16, gw1a_ref[...], preferred_element_type=f32)
    hg = hg + jnp.dot(sb16, gw1b_ref[...], preferred_element_type=f32)
    hg = jnp.maximum(hg + gb1_ref[...], 0.0)
    gate = jnp.sum(hg * gw2_ref[...], axis=1, keepdims=True) + gb2_ref[...]
    m = jax.nn.sigmoid(gate)

    hn = jnp.dot(xb16, nw1a_ref[...], preferred_element_type=f32)
    hn = hn + jnp.dot(sb16, nw1b_ref[...], preferred_element_type=f32)
    hn = jnp.maximum(hn + nb1_ref[...], 0.0).astype(bf16)
    v = jnp.dot(hn, nw2_ref[...], preferred_element_type=f32) + nb2_ref[...]
    mv = m * v

    w = w_ref[...]
    outs = []
    for b in range(_GB):
        wb = w[b * _K:(b + 1) * _K, :]
        mvb = mv[b * _K:(b + 1) * _K, :]
        outs.append(jnp.dot(wb, mvb, preferred_element_type=f32))
    agg = jnp.concatenate(outs, axis=0)

    norm = jnp.sqrt(jnp.sum(agg * agg, axis=1, keepdims=True)) + 1e-8
    images2 = xb + agg / norm

    h3 = jnp.dot(images2.astype(bf16), mw1_ref[...], preferred_element_type=f32)
    h3 = jnp.maximum(h3 + mb1_ref[...], 0.0).astype(bf16)
    emb = jnp.dot(h3, mw2_ref[...], preferred_element_type=f32) + mb2_ref[...]
    n2 = jnp.sqrt(jnp.sum(emb * emb, axis=1, keepdims=True)) + 1e-8
    out_ref[...] = emb / n2


def kernel(images, bboxes, img_range, gw1, gb1, gw2, gb2,
           nw1, nb1, nw2, nb2, mw1, mb1, mw2, mb2):
    f32 = jnp.float32
    bf16 = jnp.bfloat16

    area = (bboxes[:, :, 2] - bboxes[:, :, 0]) * (bboxes[:, :, 3] - bboxes[:, :, 1])
    s = jnp.concatenate([bboxes, area[:, :, None]], axis=2) * 0.1
    s = jnp.pad(s, ((0, 0), (0, 0), (0, 3))).reshape(_N, 8)

    imgs = images.reshape(_N, _D)
    rng2d = img_range.reshape(_N, _K)

    wmat = _sc_w_kernel(rng2d.T).T

    def _cast_then_t(wm):
        wb = jax.lax.optimization_barrier(wm.astype(bf16))
        return wb.T

    gw1a = _cast_then_t(gw1[:, :_D])
    gw1b = jnp.pad(gw1[:, _D:].T, ((0, 3), (0, 0))).astype(bf16)
    nw1a = _cast_then_t(nw1[:, :_D])
    nw1b = jnp.pad(nw1[:, _D:].T, ((0, 3), (0, 0))).astype(bf16)
    nw2t = _cast_then_t(nw2)
    mw1t = _cast_then_t(mw1)
    mw2t = _cast_then_t(mw2)

    gb1r = gb1.reshape(1, _D)
    gb2r = gb2.reshape(1, 1)
    nb1r = nb1.reshape(1, _D)
    nb2r = nb2.reshape(1, _D)
    mb1r = mb1.reshape(1, _D)
    mb2r = mb2.reshape(1, _E)

    row_spec = lambda cols: pl.BlockSpec((_RT, cols), lambda i: (i, 0))
    full_spec = lambda rows, cols: pl.BlockSpec((rows, cols), lambda i: (0, 0))

    out = pl.pallas_call(
        _tc_body,
        grid=(_B // _GB,),
        in_specs=[
            row_spec(_D),
            row_spec(8),
            row_spec(_K),
            full_spec(_D, _D),
            full_spec(8, _D),
            full_spec(1, _D),
            full_spec(1, _D),
            full_spec(1, 1),
            full_spec(_D, _D),
            full_spec(8, _D),
            full_spec(1, _D),
            full_spec(_D, _D),
            full_spec(1, _D),
            full_spec(_D, _D),
            full_spec(1, _D),
            full_spec(_D, _E),
            full_spec(1, _E),
        ],
        out_specs=row_spec(_E),
        out_shape=jax.ShapeDtypeStruct((_N, _E), f32),
        compiler_params=pltpu.CompilerParams(
            dimension_semantics=("arbitrary",),
        ),
    )(imgs, s, wmat, gw1a, gw1b, gb1r, gw2, gb2r,
      nw1a, nw1b, nb1r, nw2t, nb2r, mw1t, mb1r, mw2t, mb2r)

    return out.reshape(_B, _K, _E)


# speedup vs baseline: 1.0174x; 1.0174x over previous
import functools

import jax
import jax.numpy as jnp
from jax import lax
from jax.experimental import pallas as pl
from jax.experimental.pallas import tpu as pltpu
from jax.experimental.pallas import tpu_sc as plsc

_B, _K, _D, _E, _P = 128, 36, 2048, 1024, 5
_GB = 4
_RT = _GB * _K
_N = _B * _K

_NC, _NS, _L = 2, 16, 16
_NW = _NC * _NS
_SLABW = 128
_NSLAB = _N // _SLABW
_BLKS = _SLABW // _L


def _sc_relation_weights(rng_hbm, w_hbm, buf, sem):
    f32 = jnp.float32
    i32 = jnp.int32
    wid = lax.axis_index("s") * _NC + lax.axis_index("c")
    lanes = lax.iota(i32, _L)

    def do_slab(slab):
        base = slab * _SLABW
        cp = pltpu.make_async_copy(rng_hbm.at[:, pl.ds(base, _SLABW)],
                                   buf, sem)
        cp.start()
        cp.wait()

        def blk_body(blk, carry):
            col0 = blk * _L
            run = jnp.zeros((_L,), f32)
            for j in range(_K):
                c = buf[j, pl.ds(col0, _L)]
                run = run + c
                buf[j, pl.ds(col0, _L)] = jnp.where(run <= float(_P), c, 0.0)
            deficit = jnp.maximum(float(_P) - run, 0.0)
            dcol = jnp.remainder(base + col0 + lanes, _K)
            for j in range(_K):
                wv = buf[j, pl.ds(col0, _L)]
                buf[j, pl.ds(col0, _L)] = wv + jnp.where(dcol == j,
                                                         deficit, 0.0)
            return carry

        lax.fori_loop(0, _BLKS, blk_body, 0)

        cp = pltpu.make_async_copy(buf, w_hbm.at[:, pl.ds(base, _SLABW)], sem)
        cp.start()
        cp.wait()

    do_slab(wid)

    @pl.when(wid < _NSLAB - _NW)
    def _second_slab():
        do_slab(wid + _NW)


_sc_w_kernel = functools.partial(
    pl.kernel,
    mesh=plsc.VectorSubcoreMesh(core_axis_name="c", subcore_axis_name="s"),
    out_type=jax.ShapeDtypeStruct((_K, _N), jnp.float32),
    scratch_types=[
        pltpu.VMEM((_K, _SLABW), jnp.float32),
        pltpu.SemaphoreType.DMA,
    ],
)(_sc_relation_weights)


def _tc_body(imgs_ref, s_ref, w_ref,
             gw1a_ref, gw1b_ref, gb1_ref, gw2_ref, gb2_ref,
             nw1a_ref, nw1b_ref, nb1_ref, nw2_ref, nb2_ref,
             mw1_ref, mb1_ref, mw2_ref, mb2_ref,
             out_ref):
    f32 = jnp.float32
    bf16 = jnp.bfloat16

    xb = imgs_ref[...]
    xb16 = xb.astype(bf16)
    sb16 = s_ref[...].astype(bf16)

    hg = jnp.dot(xb16, gw1a_ref[...], preferred_element_type=f32)
    hg = hg + jnp.dot(sb16, gw1b_ref[...], preferred_element_type=f32)
    hg = jnp.maximum(hg + gb1_ref[...], 0.0)
    gate = jnp.sum(hg * gw2_ref[...], axis=1, keepdims=True) + gb2_ref[...]
    m = jax.nn.sigmoid(gate)

    hn = jnp.dot(xb16, nw1a_ref[...], preferred_element_type=f32)
    hn = hn + jnp.dot(sb16, nw1b_ref[...], preferred_element_type=f32)
    hn = jnp.maximum(hn + nb1_ref[...], 0.0).astype(bf16)
    v = jnp.dot(hn, nw2_ref[...], preferred_element_type=f32) + nb2_ref[...]
    mv = m * v

    w = w_ref[...]
    outs = []
    for b in range(_GB):
        wb = w[b * _K:(b + 1) * _K, :]
        mvb = mv[b * _K:(b + 1) * _K, :]
        outs.append(jnp.dot(wb, mvb, preferred_element_type=f32))
    agg = jnp.concatenate(outs, axis=0)

    norm = jnp.sqrt(jnp.sum(agg * agg, axis=1, keepdims=True)) + 1e-8
    images2 = xb + agg / norm

    h3 = jnp.dot(images2.astype(bf16), mw1_ref[...], preferred_element_type=f32)
    h3 = jnp.maximum(h3 + mb1_ref[...], 0.0).astype(bf16)
    emb = jnp.dot(h3, mw2_ref[...], preferred_element_type=f32) + mb2_ref[...]
    n2 = jnp.sqrt(jnp.sum(emb * emb, axis=1, keepdims=True)) + 1e-8
    out_ref[...] = emb / n2


def kernel(images, bboxes, img_range, gw1, gb1, gw2, gb2,
           nw1, nb1, nw2, nb2, mw1, mb1, mw2, mb2):
    f32 = jnp.float32
    bf16 = jnp.bfloat16

    area = (bboxes[:, :, 2] - bboxes[:, :, 0]) * (bboxes[:, :, 3] - bboxes[:, :, 1])
    s = jnp.concatenate([bboxes, area[:, :, None]], axis=2) * 0.1
    s = jnp.pad(s, ((0, 0), (0, 0), (0, 3))).reshape(_N, 8)

    imgs = images.reshape(_N, _D)
    rng2d = img_range.reshape(_N, _K)

    wmat = _sc_w_kernel(rng2d.T).T

    gw1a = gw1[:, :_D].T.astype(bf16)
    gw1b = jnp.pad(gw1[:, _D:].T, ((0, 3), (0, 0))).astype(bf16)
    nw1a = nw1[:, :_D].T.astype(bf16)
    nw1b = jnp.pad(nw1[:, _D:].T, ((0, 3), (0, 0))).astype(bf16)
    nw2t = nw2.T.astype(bf16)
    mw1t = mw1.T.astype(bf16)
    mw2t = mw2.T.astype(bf16)

    gb1r = gb1.reshape(1, _D)
    gb2r = gb2.reshape(1, 1)
    nb1r = nb1.reshape(1, _D)
    nb2r = nb2.reshape(1, _D)
    mb1r = mb1.reshape(1, _D)
    mb2r = mb2.reshape(1, _E)

    row_spec = lambda cols: pl.BlockSpec((_RT, cols), lambda i: (i, 0))
    full_spec = lambda rows, cols: pl.BlockSpec((rows, cols), lambda i: (0, 0))

    out = pl.pallas_call(
        _tc_body,
        grid=(_B // _GB,),
        in_specs=[
            row_spec(_D),
            row_spec(8),
            row_spec(_K),
            full_spec(_D, _D),
            full_spec(8, _D),
            full_spec(1, _D),
            full_spec(1, _D),
            full_spec(1, 1),
            full_spec(_D, _D),
            full_spec(8, _D),
            full_spec(1, _D),
            full_spec(_D, _D),
            full_spec(1, _D),
            full_spec(_D, _D),
            full_spec(1, _D),
            full_spec(_D, _E),
            full_spec(1, _E),
        ],
        out_specs=row_spec(_E),
        out_shape=jax.ShapeDtypeStruct((_N, _E), f32),
        compiler_params=pltpu.CompilerParams(
            dimension_semantics=("arbitrary",),
        ),
    )(imgs, s, wmat, gw1a, gw1b, gb1r, gw2, gb2r,
      nw1a, nw1b, nb1r, nw2t, nb2r, mw1t, mb1r, mw2t, mb2r)

    return out.reshape(_B, _K, _E)
